# baseline (speedup 1.0000x reference)
import jax
import jax.numpy as jnp
from jax import lax
from jax.experimental import pallas as pl
from jax.experimental.pallas import tpu as pltpu
from jax.experimental.pallas import tpu_sc as plsc

N = 100000
T = 8
D = 128

_NC = 2
_NS = 16
_NW = _NC * _NS

_CHUNK = 80
_NCHUNKS = N // _CHUNK
_SLOTS = -(-_NCHUNKS // _NW)


def _ln_table_kernel(table_ref, gamma_ref, beta_ref, out_ref):
    x = table_ref[...] * (D ** 0.5)
    mean = jnp.mean(x, axis=-1, keepdims=True)
    xc = x - mean
    var = jnp.mean(xc * xc, axis=-1, keepdims=True)
    out_ref[...] = xc * lax.rsqrt(var + 1e-5) * gamma_ref[...] + beta_ref[...]


def _normed_table(table, ln_gamma, ln_beta):
    return pl.pallas_call(
        _ln_table_kernel,
        out_shape=jax.ShapeDtypeStruct((T, D), jnp.float32),
    )(table, ln_gamma, ln_beta)


def _gather_body(ids_hbm, tab_hbm, out_hbm, idx_v, rows_v, sem):
    w = lax.axis_index("s") * _NC + lax.axis_index("c")

    def body(slot, carry):
        chunk = slot * _NW + w

        @pl.when(chunk < _NCHUNKS)
        def _():
            base = chunk * _CHUNK
            pltpu.sync_copy(ids_hbm.at[pl.ds(base, _CHUNK)], idx_v)
            pltpu.async_copy(tab_hbm.at[idx_v], rows_v, sem).wait()
            pltpu.sync_copy(rows_v, out_hbm.at[pl.ds(base, _CHUNK)])

        return carry

    lax.fori_loop(0, _SLOTS, body, 0)


def kernel(node_type_ids, table, ln_gamma, ln_beta):
    normed = _normed_table(table, ln_gamma, ln_beta)
    mesh = plsc.VectorSubcoreMesh(core_axis_name="c", subcore_axis_name="s")
    gather = pl.kernel(
        _gather_body,
        mesh=mesh,
        out_type=jax.ShapeDtypeStruct((N, D), jnp.float32),
        scratch_types=[
            pltpu.VMEM((_CHUNK,), jnp.int32),
            pltpu.VMEM((_CHUNK, D), jnp.float32),
            pltpu.SemaphoreType.DMA,
        ],
    )
    return gather(node_type_ids.astype(jnp.int32), normed)


# speedup vs baseline: 1.0135x; 1.0135x over previous
import jax
import jax.numpy as jnp
from jax import lax
from jax.experimental import pallas as pl
from jax.experimental.pallas import tpu as pltpu
from jax.experimental.pallas import tpu_sc as plsc

N = 100000
T = 8
D = 128

_NC = 2
_NS = 16
_NW = _NC * _NS

_CH = 320
_SUB = (128, 128, 64)
_NFULL = N // _CH
_TAIL = N - _NFULL * _CH
_SLOTS = -(-_NFULL // _NW)
_LAST_N = _NFULL - (_NW - 1) * _SLOTS


def _ln_table_kernel(table_ref, gamma_ref, beta_ref, out_ref):
    x = table_ref[...] * (D ** 0.5)
    mean = jnp.mean(x, axis=-1, keepdims=True)
    xc = x - mean
    var = jnp.mean(xc * xc, axis=-1, keepdims=True)
    out_ref[...] = xc * lax.rsqrt(var + 1e-5) * gamma_ref[...] + beta_ref[...]


def _normed_table(table, ln_gamma, ln_beta):
    return pl.pallas_call(
        _ln_table_kernel,
        out_shape=jax.ShapeDtypeStruct((T, D), jnp.float32),
    )(table, ln_gamma, ln_beta)


def _gather_body(ids_hbm, tab_hbm, out_hbm, ids_v, rows0, rows1, g0, g1, s0, s1):
    w = lax.axis_index("s") * _NC + lax.axis_index("c")
    c0 = w * _SLOTS
    n = jnp.minimum(_SLOTS, _NFULL - c0)
    row0 = c0 * _CH

    @pl.when(w < _NW - 1)
    def _():
        pltpu.sync_copy(ids_hbm.at[pl.ds(row0, _SLOTS * _CH)], ids_v)

    @pl.when(w == _NW - 1)
    def _():
        cnt = _LAST_N * _CH + _TAIL
        pltpu.sync_copy(ids_hbm.at[pl.ds(row0, cnt)], ids_v.at[pl.ds(0, cnt)])

    bufs = (rows0, rows1)
    gsems = (g0, g1)
    ssems = (s0, s1)

    def gather_descs(j, buf, sem):
        ds, off = [], 0
        for ln in _SUB:
            idx = ids_v.at[pl.ds(j * _CH + off, ln)]
            ds.append(pltpu.make_async_copy(
                tab_hbm.at[idx], buf.at[pl.ds(off, ln)], sem))
            off += ln
        return ds

    def store_desc(j, buf, sem):
        return pltpu.make_async_copy(
            buf, out_hbm.at[pl.ds((c0 + j) * _CH, _CH)], sem)

    def start_gather(j, buf, sem):
        for d in gather_descs(j, buf, sem):
            d.start()

    def wait_gather(j, buf, sem):
        for d in gather_descs(j, buf, sem):
            d.wait()

    start_gather(0, rows0, g0)
    start_gather(1, rows1, g1)

    def body(j, carry):
        for b in range(2):
            @pl.when(j % 2 == b)
            def _(b=b):
                wait_gather(j, bufs[b], gsems[b])
                store_desc(j, bufs[b], ssems[b]).start()

                @pl.when(j + 2 < n)
                def _():
                    store_desc(j, bufs[b], ssems[b]).wait()
                    start_gather(j + 2, bufs[b], gsems[b])

        return carry

    lax.fori_loop(0, n, body, 0)

    store_desc(n - 2, rows0, s0).wait()
    store_desc(n - 1, rows1, s1).wait()

    @pl.when(w == _NW - 1)
    def _():
        base = _LAST_N * _CH
        d1 = pltpu.make_async_copy(
            tab_hbm.at[ids_v.at[pl.ds(base, 128)]], rows0.at[pl.ds(0, 128)], g0)
        d2 = pltpu.make_async_copy(
            tab_hbm.at[ids_v.at[pl.ds(base + 128, _TAIL - 128)]],
            rows0.at[pl.ds(128, _TAIL - 128)], g0)
        d1.start()
        d2.start()
        d1.wait()
        d2.wait()
        pltpu.sync_copy(rows0.at[pl.ds(0, _TAIL)],
                        out_hbm.at[pl.ds(_NFULL * _CH, _TAIL)])


def kernel(node_type_ids, table, ln_gamma, ln_beta):
    normed = _normed_table(table, ln_gamma, ln_beta)
    mesh = plsc.VectorSubcoreMesh(core_axis_name="c", subcore_axis_name="s")
    gather = pl.kernel(
        _gather_body,
        mesh=mesh,
        out_type=jax.ShapeDtypeStruct((N, D), jnp.float32),
        scratch_types=[
            pltpu.VMEM((_SLOTS * _CH,), jnp.int32),
            pltpu.VMEM((_CH, D), jnp.float32),
            pltpu.VMEM((_CH, D), jnp.float32),
            pltpu.SemaphoreType.DMA,
            pltpu.SemaphoreType.DMA,
            pltpu.SemaphoreType.DMA,
            pltpu.SemaphoreType.DMA,
        ],
    )
    return gather(node_type_ids.astype(jnp.int32), normed)


# speedup vs baseline: 13.8395x; 13.6552x over previous
import jax
import jax.numpy as jnp
from jax import lax
from jax.experimental import pallas as pl
from jax.experimental.pallas import tpu as pltpu
from jax.experimental.pallas import tpu_sc as plsc

N = 100000
T = 8
D = 128

_NC = 2
_NS = 16
_NW = _NC * _NS

_CH = 320
_SUB = (128, 128, 64)
_NFULL = N // _CH
_TAIL = N - _NFULL * _CH
_SLOTS = -(-_NFULL // _NW)
_LAST_N = _NFULL - (_NW - 1) * _SLOTS


def _ln_table_kernel(table_ref, gamma_ref, beta_ref, out_ref):
    x = table_ref[...] * (D ** 0.5)
    mean = jnp.mean(x, axis=-1, keepdims=True)
    xc = x - mean
    var = jnp.mean(xc * xc, axis=-1, keepdims=True)
    out_ref[...] = xc * lax.rsqrt(var + 1e-5) * gamma_ref[...] + beta_ref[...]


def _normed_table(table, ln_gamma, ln_beta):
    return pl.pallas_call(
        _ln_table_kernel,
        out_shape=jax.ShapeDtypeStruct((T, D), jnp.float32),
    )(table, ln_gamma, ln_beta)


def _gather_body(ids_hbm, tab_hbm, out_hbm, tab_sp, ids_v, rows0, rows1,
                 g0, g1, s0, s1):
    w = lax.axis_index("s") * _NC + lax.axis_index("c")
    c0 = w * _SLOTS
    n = jnp.minimum(_SLOTS, _NFULL - c0)
    row0 = c0 * _CH

    @pl.when(lax.axis_index("s") == 0)
    def _():
        pltpu.sync_copy(tab_hbm, tab_sp)

    plsc.subcore_barrier()

    @pl.when(w < _NW - 1)
    def _():
        pltpu.sync_copy(ids_hbm.at[pl.ds(row0, _SLOTS * _CH)], ids_v)

    @pl.when(w == _NW - 1)
    def _():
        cnt = _LAST_N * _CH + _TAIL
        pltpu.sync_copy(ids_hbm.at[pl.ds(row0, cnt)], ids_v.at[pl.ds(0, cnt)])

    bufs = (rows0, rows1)
    gsems = (g0, g1)
    ssems = (s0, s1)

    def gather_descs(j, buf, sem):
        ds, off = [], 0
        for ln in _SUB:
            idx = ids_v.at[pl.ds(j * _CH + off, ln)]
            ds.append(pltpu.make_async_copy(
                tab_sp.at[idx], buf.at[pl.ds(off, ln)], sem))
            off += ln
        return ds

    def store_desc(j, buf, sem):
        return pltpu.make_async_copy(
            buf, out_hbm.at[pl.ds((c0 + j) * _CH, _CH)], sem)

    def start_gather(j, buf, sem):
        for d in gather_descs(j, buf, sem):
            d.start()

    def wait_gather(j, buf, sem):
        for d in gather_descs(j, buf, sem):
            d.wait()

    start_gather(0, rows0, g0)
    start_gather(1, rows1, g1)

    def body(j, carry):
        for b in range(2):
            @pl.when(j % 2 == b)
            def _(b=b):
                wait_gather(j, bufs[b], gsems[b])
                store_desc(j, bufs[b], ssems[b]).start()

                @pl.when(j + 2 < n)
                def _():
                    store_desc(j, bufs[b], ssems[b]).wait()
                    start_gather(j + 2, bufs[b], gsems[b])

        return carry

    lax.fori_loop(0, n, body, 0)

    store_desc(n - 2, rows0, s0).wait()
    store_desc(n - 1, rows1, s1).wait()

    @pl.when(w == _NW - 1)
    def _():
        base = _LAST_N * _CH
        d1 = pltpu.make_async_copy(
            tab_sp.at[ids_v.at[pl.ds(base, 128)]], rows0.at[pl.ds(0, 128)], g0)
        d2 = pltpu.make_async_copy(
            tab_sp.at[ids_v.at[pl.ds(base + 128, _TAIL - 128)]],
            rows0.at[pl.ds(128, _TAIL - 128)], g0)
        d1.start()
        d2.start()
        d1.wait()
        d2.wait()
        pltpu.sync_copy(rows0.at[pl.ds(0, _TAIL)],
                        out_hbm.at[pl.ds(_NFULL * _CH, _TAIL)])


def kernel(node_type_ids, table, ln_gamma, ln_beta):
    normed = _normed_table(table, ln_gamma, ln_beta)
    mesh = plsc.VectorSubcoreMesh(core_axis_name="c", subcore_axis_name="s")
    gather = pl.kernel(
        _gather_body,
        mesh=mesh,
        out_type=jax.ShapeDtypeStruct((N, D), jnp.float32),
        scratch_types=[
            pltpu.VMEM_SHARED((T, D), jnp.float32),
            pltpu.VMEM((_SLOTS * _CH,), jnp.int32),
            pltpu.VMEM((_CH, D), jnp.float32),
            pltpu.VMEM((_CH, D), jnp.float32),
            pltpu.SemaphoreType.DMA,
            pltpu.SemaphoreType.DMA,
            pltpu.SemaphoreType.DMA,
            pltpu.SemaphoreType.DMA,
        ],
    )
    return gather(node_type_ids.astype(jnp.int32), normed)


# speedup vs baseline: 14.2369x; 1.0287x over previous
import jax
import jax.numpy as jnp
from jax import lax
from jax.experimental import pallas as pl
from jax.experimental.pallas import tpu as pltpu
from jax.experimental.pallas import tpu_sc as plsc

N = 100000
T = 8
D = 128

_NC = 2
_NS = 16
_NW = _NC * _NS

_CH = 320
_SUB = (128, 128, 64)
_NFULL = N // _CH
_TAIL = N - _NFULL * _CH
_SLOTS = -(-_NFULL // _NW)
_LAST_N = _NFULL - (_NW - 1) * _SLOTS


def _ln_table_kernel(table_ref, gamma_ref, beta_ref, out_ref):
    x = table_ref[...] * (D ** 0.5)
    mean = jnp.mean(x, axis=-1, keepdims=True)
    xc = x - mean
    var = jnp.mean(xc * xc, axis=-1, keepdims=True)
    out_ref[...] = xc * lax.rsqrt(var + 1e-5) * gamma_ref[...] + beta_ref[...]


def _normed_table(table, ln_gamma, ln_beta):
    return pl.pallas_call(
        _ln_table_kernel,
        out_shape=jax.ShapeDtypeStruct((T, D), jnp.float32),
    )(table, ln_gamma, ln_beta)


_B = 3


def _gather_body(ids_hbm, tab_hbm, out_hbm, tab_sp, ids_v, rows0, rows1,
                 rows2, g0, g1, g2, s0, s1, s2):
    w = lax.axis_index("s") * _NC + lax.axis_index("c")
    c0 = w * _SLOTS
    n = jnp.minimum(_SLOTS, _NFULL - c0)
    row0 = c0 * _CH

    @pl.when(lax.axis_index("s") == 0)
    def _():
        pltpu.sync_copy(tab_hbm, tab_sp)

    plsc.subcore_barrier()

    @pl.when(w < _NW - 1)
    def _():
        pltpu.sync_copy(ids_hbm.at[pl.ds(row0, _SLOTS * _CH)], ids_v)

    @pl.when(w == _NW - 1)
    def _():
        cnt = _LAST_N * _CH + _TAIL
        pltpu.sync_copy(ids_hbm.at[pl.ds(row0, cnt)], ids_v.at[pl.ds(0, cnt)])

    bufs = (rows0, rows1, rows2)
    gsems = (g0, g1, g2)
    ssems = (s0, s1, s2)

    def gather_descs(j, buf, sem):
        ds, off = [], 0
        for ln in _SUB:
            idx = ids_v.at[pl.ds(j * _CH + off, ln)]
            ds.append(pltpu.make_async_copy(
                tab_sp.at[idx], buf.at[pl.ds(off, ln)], sem))
            off += ln
        return ds

    def store_desc(j, buf, sem):
        return pltpu.make_async_copy(
            buf, out_hbm.at[pl.ds((c0 + j) * _CH, _CH)], sem)

    def start_gather(j, buf, sem):
        for d in gather_descs(j, buf, sem):
            d.start()

    def wait_gather(j, buf, sem):
        for d in gather_descs(j, buf, sem):
            d.wait()

    start_gather(0, rows0, g0)
    start_gather(1, rows1, g1)

    @pl.when(n > 2)
    def _():
        start_gather(2, rows2, g2)

    def body(j, carry):
        for b in range(_B):
            @pl.when(j % _B == b)
            def _(b=b):
                wait_gather(j, bufs[b], gsems[b])
                store_desc(j, bufs[b], ssems[b]).start()

                @pl.when(j + _B < n)
                def _():
                    store_desc(j, bufs[b], ssems[b]).wait()
                    start_gather(j + _B, bufs[b], gsems[b])

        return carry

    lax.fori_loop(0, n, body, 0)

    def drain(j, carry):
        for b in range(_B):
            @pl.when(j % _B == b)
            def _(b=b):
                store_desc(j, bufs[b], ssems[b]).wait()

        return carry

    lax.fori_loop(jnp.maximum(n - _B, 0), n, drain, 0)

    @pl.when(w == _NW - 1)
    def _():
        base = _LAST_N * _CH
        d1 = pltpu.make_async_copy(
            tab_sp.at[ids_v.at[pl.ds(base, 128)]], rows0.at[pl.ds(0, 128)], g0)
        d2 = pltpu.make_async_copy(
            tab_sp.at[ids_v.at[pl.ds(base + 128, _TAIL - 128)]],
            rows0.at[pl.ds(128, _TAIL - 128)], g0)
        d1.start()
        d2.start()
        d1.wait()
        d2.wait()
        pltpu.sync_copy(rows0.at[pl.ds(0, _TAIL)],
                        out_hbm.at[pl.ds(_NFULL * _CH, _TAIL)])


def kernel(node_type_ids, table, ln_gamma, ln_beta):
    normed = _normed_table(table, ln_gamma, ln_beta)
    mesh = plsc.VectorSubcoreMesh(core_axis_name="c", subcore_axis_name="s")
    gather = pl.kernel(
        _gather_body,
        mesh=mesh,
        out_type=jax.ShapeDtypeStruct((N, D), jnp.float32),
        scratch_types=[
            pltpu.VMEM_SHARED((T, D), jnp.float32),
            pltpu.VMEM((_SLOTS * _CH,), jnp.int32),
            pltpu.VMEM((_CH, D), jnp.float32),
            pltpu.VMEM((_CH, D), jnp.float32),
            pltpu.VMEM((_CH, D), jnp.float32),
            pltpu.SemaphoreType.DMA,
            pltpu.SemaphoreType.DMA,
            pltpu.SemaphoreType.DMA,
            pltpu.SemaphoreType.DMA,
            pltpu.SemaphoreType.DMA,
            pltpu.SemaphoreType.DMA,
        ],
    )
    return gather(node_type_ids.astype(jnp.int32), normed)
